# Initial kernel scaffold; baseline (speedup 1.0000x reference)
import jax
import jax.numpy as jnp
from jax.experimental import pallas as pl


def kernel(x, edge_index, W1, b1, W2, b2):
    raise NotImplementedError("write your pallas kernel here")


# speedup vs baseline: 10.4438x; 10.4438x over previous
import functools

import jax
import jax.numpy as jnp
from jax import lax
from jax.experimental import pallas as pl
from jax.experimental.pallas import tpu as pltpu
from jax.experimental.pallas import tpu_sc as plsc

N = 10000
E = 320000
D = 128
NC = 2
NS = 16
NW = NC * NS
LANES = 16

NPAD = 10240
EPAD = 327680
EPT = EPAD // NW
CHUNKS = EPT // D
RPT = NPAD // NS

_mesh = plsc.VectorSubcoreMesh(
    core_axis_name="c", subcore_axis_name="s", num_cores=NC, num_subcores=NS
)
_sc_params = pltpu.CompilerParams(needs_layout_passes=False)


@functools.partial(
    pl.kernel,
    out_type=jax.ShapeDtypeStruct((NW, NPAD), jnp.float32),
    mesh=_mesh,
    scratch_types=[
        pltpu.VMEM((EPT,), jnp.int32),
        pltpu.VMEM((NPAD,), jnp.float32),
    ],
    compiler_params=_sc_params,
)
def _deg_kernel(dst_hbm, out_hbm, dst_v, hist_v):
    c = lax.axis_index("c")
    s = lax.axis_index("s")
    wid = c * NS + s
    pltpu.sync_copy(dst_hbm.at[pl.ds(wid * EPT, EPT)], dst_v)

    def zbody(i, carry):
        hist_v[pl.ds(i * LANES, LANES)] = jnp.zeros((LANES,), jnp.float32)
        return carry

    lax.fori_loop(0, NPAD // LANES, zbody, 0)

    ones = jnp.ones((LANES,), jnp.float32)

    def abody(i, carry):
        idx = dst_v[pl.ds(i * LANES, LANES)]
        plsc.addupdate_scatter(hist_v, [idx], ones)
        return carry

    lax.fori_loop(0, EPT // LANES, abody, 0)
    pltpu.sync_copy(hist_v, out_hbm.at[wid])


@functools.partial(
    pl.kernel,
    out_type=jax.ShapeDtypeStruct((NC, NPAD, D), jnp.float32),
    mesh=_mesh,
    scratch_types=[
        pltpu.VMEM((CHUNKS, D), jnp.int32),
        pltpu.VMEM((CHUNKS, D), jnp.int32),
        pltpu.VMEM((D, D), jnp.float32),
        pltpu.VMEM_SHARED((NPAD, D), jnp.float32),
        pltpu.SemaphoreType.DMA,
    ],
    compiler_params=_sc_params,
)
def _agg_kernel(ytab_hbm, src_hbm, dst_hbm, out_hbm, src_v, dst_v, rows_v,
                acc, sem):
    c = lax.axis_index("c")
    s = lax.axis_index("s")
    wid = c * NS + s

    pltpu.sync_copy(src_hbm.at[pl.ds(wid * CHUNKS, CHUNKS)], src_v)
    pltpu.sync_copy(dst_hbm.at[pl.ds(wid * CHUNKS, CHUNKS)], dst_v)

    for r in range(D):
        for q in range(D // LANES):
            rows_v[r, pl.ds(q * LANES, LANES)] = jnp.zeros(
                (LANES,), jnp.float32)
    for k in range(RPT // D):
        pltpu.sync_copy(rows_v, acc.at[pl.ds(s * RPT + k * D, D)])
    plsc.subcore_barrier()

    def body(b, carry):
        pltpu.async_copy(ytab_hbm.at[src_v.at[b]], rows_v, sem).wait()
        pltpu.sync_copy(rows_v, acc.at[dst_v.at[b]], add=True)
        return carry

    lax.fori_loop(0, CHUNKS, body, 0)
    plsc.subcore_barrier()
    pltpu.sync_copy(acc.at[pl.ds(s * RPT, RPT)],
                    out_hbm.at[c, pl.ds(s * RPT, RPT)])


def _dinv_from_parts(degp):
    deg = jnp.sum(degp, axis=0) + 1.0
    return lax.rsqrt(deg).reshape(NPAD, 1)


def _tc1_body(degp_ref, x_ref, w_ref, ytab_ref):
    dinv = _dinv_from_parts(degp_ref[...])
    xw = jnp.dot(x_ref[...], w_ref[...], preferred_element_type=jnp.float32)
    ytab_ref[...] = dinv * xw


def _tc2_body(degp_ref, accp_ref, ytab1_ref, b1_ref, w_ref, ytab2_ref):
    dinv = _dinv_from_parts(degp_ref[...])
    accsum = accp_ref[0] + accp_ref[1] + ytab1_ref[...]
    h = dinv * accsum + b1_ref[...]
    hw = jnp.dot(h, w_ref[...], preferred_element_type=jnp.float32)
    ytab2_ref[...] = dinv * hw


def _tc3_body(degp_ref, accp_ref, ytab2_ref, b2_ref, out_ref):
    dinv = _dinv_from_parts(degp_ref[...])
    accsum = accp_ref[0] + accp_ref[1] + ytab2_ref[...]
    out_ref[...] = dinv * accsum + b2_ref[...]


_f32 = jnp.float32


def kernel(x, edge_index, W1, b1, W2, b2):
    src = edge_index[0].astype(jnp.int32)
    dst = edge_index[1].astype(jnp.int32)
    src_p = jnp.concatenate(
        [src, jnp.zeros((EPAD - E,), jnp.int32)]).reshape(EPAD // D, D)
    dst_p = jnp.concatenate(
        [dst, jnp.full((EPAD - E,), N, jnp.int32)])
    dst_p2 = dst_p.reshape(EPAD // D, D)

    x_p = jnp.pad(x, ((0, NPAD - N), (0, 0)))
    b1r = b1.reshape(1, D)
    b2r = b2.reshape(1, D)

    degp = _deg_kernel(dst_p)

    ytab1 = pl.pallas_call(
        _tc1_body,
        out_shape=jax.ShapeDtypeStruct((NPAD, D), _f32),
    )(degp, x_p, W1)

    acc1 = _agg_kernel(ytab1, src_p, dst_p2)

    ytab2 = pl.pallas_call(
        _tc2_body,
        out_shape=jax.ShapeDtypeStruct((NPAD, D), _f32),
    )(degp, acc1, ytab1, b1r, W2)

    acc2 = _agg_kernel(ytab2, src_p, dst_p2)

    out_p = pl.pallas_call(
        _tc3_body,
        out_shape=jax.ShapeDtypeStruct((NPAD, D), _f32),
    )(degp, acc2, ytab2, b2r)

    return out_p[:N]


# speedup vs baseline: 10.4484x; 1.0004x over previous
import functools

import jax
import jax.numpy as jnp
from jax import lax
from jax.experimental import pallas as pl
from jax.experimental.pallas import tpu as pltpu
from jax.experimental.pallas import tpu_sc as plsc

N = 10000
E = 320000
D = 128
NC = 2
NS = 16
NW = NC * NS
LANES = 16

NPAD = 10240
EPAD = 327680
EPT = EPAD // NW
CHUNKS = EPT // D
RPT = NPAD // NS

_mesh = plsc.VectorSubcoreMesh(
    core_axis_name="c", subcore_axis_name="s", num_cores=NC, num_subcores=NS
)
_sc_params = pltpu.CompilerParams(needs_layout_passes=False)


@functools.partial(
    pl.kernel,
    out_type=jax.ShapeDtypeStruct((NW, NPAD), jnp.float32),
    mesh=_mesh,
    scratch_types=[
        pltpu.VMEM((EPT,), jnp.int32),
        pltpu.VMEM((NPAD,), jnp.float32),
    ],
    compiler_params=_sc_params,
)
def _deg_kernel(dst_hbm, out_hbm, dst_v, hist_v):
    c = lax.axis_index("c")
    s = lax.axis_index("s")
    wid = c * NS + s
    pltpu.sync_copy(dst_hbm.at[pl.ds(wid * EPT, EPT)], dst_v)

    def zbody(i, carry):
        hist_v[pl.ds(i * LANES, LANES)] = jnp.zeros((LANES,), jnp.float32)
        return carry

    lax.fori_loop(0, NPAD // LANES, zbody, 0)

    ones = jnp.ones((LANES,), jnp.float32)

    def abody(i, carry):
        idx = dst_v[pl.ds(i * LANES, LANES)]
        plsc.addupdate_scatter(hist_v, [idx], ones)
        return carry

    lax.fori_loop(0, EPT // LANES, abody, 0)
    pltpu.sync_copy(hist_v, out_hbm.at[wid])


@functools.partial(
    pl.kernel,
    out_type=jax.ShapeDtypeStruct((NC, NPAD, D), jnp.float32),
    mesh=_mesh,
    scratch_types=[
        pltpu.VMEM((CHUNKS, D), jnp.int32),
        pltpu.VMEM((CHUNKS, D), jnp.int32),
        pltpu.VMEM((D, D), jnp.float32),
        pltpu.VMEM_SHARED((NPAD, D), jnp.float32),
        pltpu.SemaphoreType.DMA,
    ],
    compiler_params=_sc_params,
)
def _agg_kernel(ytab_hbm, src_hbm, dst_hbm, out_hbm, src_v, dst_v, rows_v,
                acc, gsem):
    c = lax.axis_index("c")
    s = lax.axis_index("s")
    wid = c * NS + s

    pltpu.sync_copy(src_hbm.at[pl.ds(wid * CHUNKS, CHUNKS)], src_v)
    pltpu.sync_copy(dst_hbm.at[pl.ds(wid * CHUNKS, CHUNKS)], dst_v)

    for r in range(D):
        for q in range(D // LANES):
            rows_v[r, pl.ds(q * LANES, LANES)] = jnp.zeros(
                (LANES,), jnp.float32)
    for k in range(RPT // D):
        pltpu.sync_copy(rows_v,
                        acc.at[pl.ds(s * RPT + k * D, D)])
    plsc.subcore_barrier()

    def body(b, carry):
        pltpu.async_copy(
            ytab_hbm.at[src_v.at[b]], rows_v, gsem).wait()
        pltpu.sync_copy(rows_v, acc.at[dst_v.at[b]], add=True)
        return carry

    lax.fori_loop(0, CHUNKS, body, 0)
    plsc.subcore_barrier()
    pltpu.sync_copy(acc.at[pl.ds(s * RPT, RPT)],
                    out_hbm.at[c, pl.ds(s * RPT, RPT)])


def _dinv_from_parts(degp):
    deg = jnp.sum(degp, axis=0) + 1.0
    return lax.rsqrt(deg).reshape(NPAD, 1)


def _tc1_body(degp_ref, x_ref, w_ref, ytab_ref):
    dinv = _dinv_from_parts(degp_ref[...])
    xw = jnp.dot(x_ref[...], w_ref[...], preferred_element_type=jnp.float32)
    ytab_ref[...] = dinv * xw


def _tc2_body(degp_ref, accp_ref, ytab1_ref, b1_ref, w_ref, ytab2_ref):
    dinv = _dinv_from_parts(degp_ref[...])
    accsum = accp_ref[0] + accp_ref[1] + ytab1_ref[...]
    h = dinv * accsum + b1_ref[...]
    hw = jnp.dot(h, w_ref[...], preferred_element_type=jnp.float32)
    ytab2_ref[...] = dinv * hw


def _tc3_body(degp_ref, accp_ref, ytab2_ref, b2_ref, out_ref):
    dinv = _dinv_from_parts(degp_ref[...])
    accsum = accp_ref[0] + accp_ref[1] + ytab2_ref[...]
    out_ref[...] = dinv * accsum + b2_ref[...]


_f32 = jnp.float32


def kernel(x, edge_index, W1, b1, W2, b2):
    src = edge_index[0].astype(jnp.int32)
    dst = edge_index[1].astype(jnp.int32)
    src_p = jnp.concatenate(
        [src, jnp.zeros((EPAD - E,), jnp.int32)]).reshape(EPAD // D, D)
    dst_p = jnp.concatenate(
        [dst, jnp.full((EPAD - E,), N, jnp.int32)])
    dst_p2 = dst_p.reshape(EPAD // D, D)

    x_p = jnp.pad(x, ((0, NPAD - N), (0, 0)))
    b1r = b1.reshape(1, D)
    b2r = b2.reshape(1, D)

    degp = _deg_kernel(dst_p)

    ytab1 = pl.pallas_call(
        _tc1_body,
        out_shape=jax.ShapeDtypeStruct((NPAD, D), _f32),
    )(degp, x_p, W1)

    acc1 = _agg_kernel(ytab1, src_p, dst_p2)

    ytab2 = pl.pallas_call(
        _tc2_body,
        out_shape=jax.ShapeDtypeStruct((NPAD, D), _f32),
    )(degp, acc1, ytab1, b1r, W2)

    acc2 = _agg_kernel(ytab2, src_p, dst_p2)

    out_p = pl.pallas_call(
        _tc3_body,
        out_shape=jax.ShapeDtypeStruct((NPAD, D), _f32),
    )(degp, acc2, ytab2, b2r)

    return out_p[:N]


# speedup vs baseline: 24.0747x; 2.3041x over previous
import functools

import jax
import jax.numpy as jnp
from jax import lax
from jax.experimental import pallas as pl
from jax.experimental.pallas import tpu as pltpu
from jax.experimental.pallas import tpu_sc as plsc

N = 10000
E = 320000
D = 128
NC = 2
NS = 16
NW = NC * NS
LANES = 16

NPAD = 10240
EPAD = 327680
EPT = EPAD // NW
CHUNKS = EPT // D
RPT = NPAD // NS

_mesh = plsc.VectorSubcoreMesh(
    core_axis_name="c", subcore_axis_name="s", num_cores=NC, num_subcores=NS
)
_sc_params = pltpu.CompilerParams(needs_layout_passes=False)


@functools.partial(
    pl.kernel,
    out_type=jax.ShapeDtypeStruct((NW, NPAD), jnp.float32),
    mesh=_mesh,
    scratch_types=[
        pltpu.VMEM((EPT,), jnp.int32),
        pltpu.VMEM((NPAD,), jnp.float32),
    ],
    compiler_params=_sc_params,
)
def _deg_kernel(dst_hbm, out_hbm, dst_v, hist_v):
    c = lax.axis_index("c")
    s = lax.axis_index("s")
    wid = c * NS + s
    pltpu.sync_copy(dst_hbm.at[pl.ds(wid * EPT, EPT)], dst_v)

    def zbody(i, carry):
        hist_v[pl.ds(i * LANES, LANES)] = jnp.zeros((LANES,), jnp.float32)
        return carry

    lax.fori_loop(0, NPAD // LANES, zbody, 0)

    ones = jnp.ones((LANES,), jnp.float32)

    def abody(i, carry):
        idx = dst_v[pl.ds(i * LANES, LANES)]
        plsc.addupdate_scatter(hist_v, [idx], ones)
        return carry

    lax.fori_loop(0, EPT // LANES, abody, 0)
    pltpu.sync_copy(hist_v, out_hbm.at[wid])


@functools.partial(
    pl.kernel,
    out_type=jax.ShapeDtypeStruct((NC, NPAD, D), jnp.float32),
    mesh=_mesh,
    scratch_types=[
        pltpu.VMEM((CHUNKS, D), jnp.int32),
        pltpu.VMEM((CHUNKS, D), jnp.int32),
        pltpu.VMEM((D, D), jnp.float32),
        pltpu.VMEM_SHARED((NPAD, D), jnp.float32),
        pltpu.SemaphoreType.DMA,
    ],
    compiler_params=_sc_params,
)
def _agg_kernel(ytab_hbm, src_hbm, dst_hbm, out_hbm, src_v, dst_v, rows_v,
                acc, gsem):
    c = lax.axis_index("c")
    s = lax.axis_index("s")
    wid = c * NS + s

    pltpu.sync_copy(src_hbm.at[pl.ds(wid * CHUNKS, CHUNKS)], src_v)
    pltpu.sync_copy(dst_hbm.at[pl.ds(wid * CHUNKS, CHUNKS)], dst_v)

    for r in range(D):
        for q in range(D // LANES):
            rows_v[r, pl.ds(q * LANES, LANES)] = jnp.zeros(
                (LANES,), jnp.float32)
    for k in range(RPT // D):
        pltpu.sync_copy(rows_v,
                        acc.at[pl.ds(s * RPT + k * D, D)])
    plsc.subcore_barrier()

    def body(b, carry):
        pltpu.async_copy(
            ytab_hbm.at[src_v.at[b]], rows_v, gsem).wait()
        pltpu.sync_copy(rows_v, acc.at[dst_v.at[b]], add=True)
        return carry

    lax.fori_loop(0, CHUNKS, body, 0)
    plsc.subcore_barrier()
    pltpu.sync_copy(acc.at[pl.ds(s * RPT, RPT)],
                    out_hbm.at[c, pl.ds(s * RPT, RPT)])


def _dinv_from_parts(degp):
    deg = jnp.sum(degp, axis=0) + 1.0
    return lax.rsqrt(deg).reshape(NPAD, 1)


def _tc1_body(degp_ref, x_ref, w_ref, ytab_ref):
    dinv = _dinv_from_parts(degp_ref[...])
    xw = jnp.dot(x_ref[...], w_ref[...], preferred_element_type=jnp.float32)
    ytab_ref[...] = dinv * xw


def _tc2_body(degp_ref, accp_ref, ytab1_ref, b1_ref, w_ref, ytab2_ref):
    dinv = _dinv_from_parts(degp_ref[...])
    accsum = accp_ref[0] + accp_ref[1] + ytab1_ref[...]
    h = dinv * accsum + b1_ref[...]
    hw = jnp.dot(h, w_ref[...], preferred_element_type=jnp.float32)
    ytab2_ref[...] = dinv * hw


def _tc3_body(degp_ref, accp_ref, ytab2_ref, b2_ref, out_ref):
    dinv = _dinv_from_parts(degp_ref[...])
    accsum = accp_ref[0] + accp_ref[1] + ytab2_ref[...]
    out_ref[...] = dinv * accsum + b2_ref[...]


_f32 = jnp.float32


def kernel(x, edge_index, W1, b1, W2, b2):
    src = edge_index[0].astype(jnp.int32)
    dst = edge_index[1].astype(jnp.int32)
    junk = N + (jnp.arange(EPAD - E, dtype=jnp.int32) % (NPAD - N))
    src_p = jnp.concatenate([src, junk]).reshape(EPAD // D, D)
    dst_p = jnp.concatenate([dst, junk])
    dst_p2 = dst_p.reshape(EPAD // D, D)

    x_p = jnp.pad(x, ((0, NPAD - N), (0, 0)))
    b1r = b1.reshape(1, D)
    b2r = b2.reshape(1, D)

    degp = _deg_kernel(dst_p)

    ytab1 = pl.pallas_call(
        _tc1_body,
        out_shape=jax.ShapeDtypeStruct((NPAD, D), _f32),
    )(degp, x_p, W1)

    acc1 = _agg_kernel(ytab1, src_p, dst_p2)

    ytab2 = pl.pallas_call(
        _tc2_body,
        out_shape=jax.ShapeDtypeStruct((NPAD, D), _f32),
    )(degp, acc1, ytab1, b1r, W2)

    acc2 = _agg_kernel(ytab2, src_p, dst_p2)

    out_p = pl.pallas_call(
        _tc3_body,
        out_shape=jax.ShapeDtypeStruct((NPAD, D), _f32),
    )(degp, acc2, ytab2, b2r)

    return out_p[:N]
